# baseline (speedup 1.0000x reference)
import jax
import jax.numpy as jnp
from jax import lax
from jax.experimental import pallas as pl
from jax.experimental.pallas import tpu as pltpu
from jax.experimental.pallas import tpu_sc as plsc

DIM = 64
B0, B1 = 16384, 26
NUM_WORKERS = 32
ROWS_W = B0 // NUM_WORKERS
DEPTH = 4
N_GROUPS = ROWS_W // DEPTH


def _gather_body(table_hbm, x_hbm, out_hbm, xv, gs, sems, wsems):
    wid = lax.axis_index("s") * 2 + lax.axis_index("c")
    brow = pl.multiple_of(wid * ROWS_W, ROWS_W)
    pltpu.sync_copy(x_hbm.at[pl.ds(brow, ROWS_W), :], xv)

    def start_gather(r, k):
        pltpu.async_copy(table_hbm.at[xv.at[r, :]], gs.at[k], sems.at[k])

    def wait_gather(k):
        pltpu.make_async_copy(
            table_hbm.at[pl.ds(0, B1)], gs.at[k], sems.at[k]
        ).wait()

    def write(r, k):
        pltpu.async_copy(gs.at[k], out_hbm.at[brow + r], wsems.at[k])

    def wait_write(k):
        pltpu.make_async_copy(gs.at[k], out_hbm.at[brow], wsems.at[k]).wait()

    for k in range(DEPTH):
        start_gather(k, k)

    def body(g, carry):
        r0 = g * DEPTH
        for k in range(DEPTH):
            wait_gather(k)
            write(r0 + k, k)

        @pl.when(g < N_GROUPS - 1)
        def _():
            for k in range(DEPTH):
                wait_write(k)
                start_gather(r0 + DEPTH + k, k)

        return carry

    lax.fori_loop(0, N_GROUPS, body, 0)
    for k in range(DEPTH):
        wait_write(k)


def kernel(X, embeddings):
    mesh = plsc.VectorSubcoreMesh(core_axis_name="c", subcore_axis_name="s")
    out = pl.kernel(
        _gather_body,
        out_type=jax.ShapeDtypeStruct((B0, B1, DIM), jnp.float32),
        mesh=mesh,
        scratch_types=[
            pltpu.VMEM((ROWS_W, B1), jnp.int32),
            pltpu.VMEM((DEPTH, B1, DIM), jnp.float32),
            pltpu.SemaphoreType.DMA((DEPTH,)),
            pltpu.SemaphoreType.DMA((DEPTH,)),
        ],
        compiler_params=pltpu.CompilerParams(use_tc_tiling_on_sc=False),
    )(embeddings, X)
    return out


# speedup vs baseline: 1.0956x; 1.0956x over previous
import jax
import jax.numpy as jnp
from jax import lax
from jax.experimental import pallas as pl
from jax.experimental.pallas import tpu as pltpu
from jax.experimental.pallas import tpu_sc as plsc

DIM = 64
PAD = 128
B0, B1 = 16384, 26
B_TOTAL = B0 * B1
NUM_WORKERS = 32
PER_W = B_TOTAL // NUM_WORKERS
CHUNK = 256
N_CHUNKS = PER_W // CHUNK
N_PAIRS = N_CHUNKS // 2


def _gather_body(table_hbm, idx_hbm, out_hbm, idx_v, rows0, rows1, sem0, sem1):
    wid = lax.axis_index("s") * 2 + lax.axis_index("c")
    base = pl.multiple_of(wid * PER_W, PER_W)
    pltpu.sync_copy(idx_hbm.at[pl.ds(base, PER_W)], idx_v)

    def start_gather(i, rows, sem):
        off = pl.multiple_of(i * CHUNK, CHUNK)
        pltpu.async_copy(table_hbm.at[idx_v.at[pl.ds(off, CHUNK)]], rows, sem)

    def wait_gather(rows, sem):
        pltpu.make_async_copy(table_hbm.at[pl.ds(0, CHUNK)], rows, sem).wait()

    def write(i, rows):
        off = pl.multiple_of(i * CHUNK, CHUNK)
        pltpu.sync_copy(
            rows.at[:, pl.ds(0, DIM)],
            out_hbm.at[pl.ds(base + off, CHUNK)],
        )

    start_gather(0, rows0, sem0)

    def pair_body(p, carry):
        i0 = p * 2
        start_gather(i0 + 1, rows1, sem1)
        wait_gather(rows0, sem0)
        write(i0, rows0)

        @pl.when(p < N_PAIRS - 1)
        def _():
            start_gather(i0 + 2, rows0, sem0)

        wait_gather(rows1, sem1)
        write(i0 + 1, rows1)
        return carry

    lax.fori_loop(0, N_PAIRS, pair_body, 0)


def kernel(X, embeddings):
    idx = X.reshape(-1)
    table = jnp.pad(embeddings, ((0, 0), (0, PAD - DIM)))
    mesh = plsc.VectorSubcoreMesh(core_axis_name="c", subcore_axis_name="s")
    out = pl.kernel(
        _gather_body,
        out_type=jax.ShapeDtypeStruct((B_TOTAL, DIM), jnp.float32),
        mesh=mesh,
        scratch_types=[
            pltpu.VMEM((PER_W,), jnp.int32),
            pltpu.VMEM((CHUNK, PAD), jnp.float32),
            pltpu.VMEM((CHUNK, PAD), jnp.float32),
            pltpu.SemaphoreType.DMA,
            pltpu.SemaphoreType.DMA,
        ],
        compiler_params=pltpu.CompilerParams(use_tc_tiling_on_sc=False),
    )(table, idx)
    return out.reshape(B0, B1, DIM)


# speedup vs baseline: 1.1473x; 1.0472x over previous
import jax
import jax.numpy as jnp
from jax import lax
from jax.experimental import pallas as pl
from jax.experimental.pallas import tpu as pltpu
from jax.experimental.pallas import tpu_sc as plsc

DIM = 64
PAD = 128
B0, B1 = 16384, 26
B_TOTAL = B0 * B1
NUM_WORKERS = 32
PER_W = B_TOTAL // NUM_WORKERS
CHUNK = 512
N_CHUNKS = PER_W // CHUNK
N_PAIRS = N_CHUNKS // 2


def _gather_body(table_hbm, idx_hbm, out_hbm, idx_v, rows0, rows1, sem0, sem1):
    wid = lax.axis_index("s") * 2 + lax.axis_index("c")
    base = pl.multiple_of(wid * PER_W, PER_W)
    pltpu.sync_copy(idx_hbm.at[pl.ds(base, PER_W)], idx_v)

    def start_gather(i, rows, sem):
        off = pl.multiple_of(i * CHUNK, CHUNK)
        pltpu.async_copy(table_hbm.at[idx_v.at[pl.ds(off, CHUNK)]], rows, sem)

    def wait_gather(rows, sem):
        pltpu.make_async_copy(table_hbm.at[pl.ds(0, CHUNK)], rows, sem).wait()

    def write(i, rows):
        off = pl.multiple_of(i * CHUNK, CHUNK)
        pltpu.sync_copy(rows, out_hbm.at[pl.ds(base + off, CHUNK)])

    start_gather(0, rows0, sem0)

    def pair_body(p, carry):
        i0 = p * 2
        start_gather(i0 + 1, rows1, sem1)
        wait_gather(rows0, sem0)
        write(i0, rows0)

        @pl.when(p < N_PAIRS - 1)
        def _():
            start_gather(i0 + 2, rows0, sem0)

        wait_gather(rows1, sem1)
        write(i0 + 1, rows1)
        return carry

    lax.fori_loop(0, N_PAIRS, pair_body, 0)


def kernel(X, embeddings):
    idx = X.reshape(-1) * 2
    table = jnp.pad(embeddings, ((0, 0), (0, PAD - DIM))).reshape(
        2 * embeddings.shape[0], DIM
    )
    mesh = plsc.VectorSubcoreMesh(core_axis_name="c", subcore_axis_name="s")
    out = pl.kernel(
        _gather_body,
        out_type=jax.ShapeDtypeStruct((B_TOTAL, DIM), jnp.float32),
        mesh=mesh,
        scratch_types=[
            pltpu.VMEM((PER_W,), jnp.int32),
            pltpu.VMEM((CHUNK, DIM), jnp.float32),
            pltpu.VMEM((CHUNK, DIM), jnp.float32),
            pltpu.SemaphoreType.DMA,
            pltpu.SemaphoreType.DMA,
        ],
        compiler_params=pltpu.CompilerParams(use_tc_tiling_on_sc=False),
    )(table, idx)
    return out.reshape(B0, B1, DIM)


# speedup vs baseline: 1.2976x; 1.1310x over previous
import jax
import jax.numpy as jnp
from jax import lax
from jax.experimental import pallas as pl
from jax.experimental.pallas import tpu as pltpu
from jax.experimental.pallas import tpu_sc as plsc

DIM = 64
PAD = 128
B0, B1 = 16384, 26
B1P = 32
B_TOTAL = B0 * B1
NUM_WORKERS = 32
PER_W = B_TOTAL // NUM_WORKERS
CHUNK = 256
N_CHUNKS = PER_W // CHUNK
N_PAIRS = N_CHUNKS // 2


def _gather_body(table_hbm, idx_hbm, dst_hbm, out_hbm,
                 idx_v, dst_v, rows0, rows1, sem0, sem1):
    wid = lax.axis_index("s") * 2 + lax.axis_index("c")
    base = pl.multiple_of(wid * PER_W, PER_W)
    row_base = pl.multiple_of(wid * N_CHUNKS, N_CHUNKS)
    pltpu.sync_copy(idx_hbm.at[pl.ds(base, PER_W)], idx_v)
    pltpu.sync_copy(dst_hbm.at[pl.ds(row_base, N_CHUNKS)], dst_v)

    def start_gather(i, rows, sem):
        off = pl.multiple_of(i * CHUNK, CHUNK)
        pltpu.async_copy(table_hbm.at[idx_v.at[pl.ds(off, CHUNK)]], rows, sem)

    def wait_gather(rows, sem):
        pltpu.make_async_copy(table_hbm.at[pl.ds(0, CHUNK)], rows, sem).wait()

    def write(i, rows):
        pltpu.sync_copy(rows, out_hbm.at[dst_v.at[i]])

    start_gather(0, rows0, sem0)

    def pair_body(p, carry):
        i0 = p * 2
        start_gather(i0 + 1, rows1, sem1)
        wait_gather(rows0, sem0)
        write(i0, rows0)

        @pl.when(p < N_PAIRS - 1)
        def _():
            start_gather(i0 + 2, rows0, sem0)

        wait_gather(rows1, sem1)
        write(i0 + 1, rows1)
        return carry

    lax.fori_loop(0, N_PAIRS, pair_body, 0)


def kernel(X, embeddings):
    idx = X.reshape(-1)
    f = jnp.arange(B_TOTAL, dtype=jnp.int32)
    dst = (f + 6 * (f // B1)).reshape(B_TOTAL // CHUNK, CHUNK)
    table = jnp.pad(embeddings, ((0, 0), (0, PAD - DIM)))
    mesh = plsc.VectorSubcoreMesh(core_axis_name="c", subcore_axis_name="s")
    out = pl.kernel(
        _gather_body,
        out_type=jax.ShapeDtypeStruct((B0 * B1P, PAD), jnp.float32),
        mesh=mesh,
        scratch_types=[
            pltpu.VMEM((PER_W,), jnp.int32),
            pltpu.VMEM((N_CHUNKS, CHUNK), jnp.int32),
            pltpu.VMEM((CHUNK, PAD), jnp.float32),
            pltpu.VMEM((CHUNK, PAD), jnp.float32),
            pltpu.SemaphoreType.DMA,
            pltpu.SemaphoreType.DMA,
        ],
        compiler_params=pltpu.CompilerParams(use_tc_tiling_on_sc=False),
    )(table, idx, dst)
    return out.reshape(B0, B1P, PAD)[:, :B1, :DIM]
